# Initial kernel scaffold; baseline (speedup 1.0000x reference)
import jax
import jax.numpy as jnp
from jax.experimental import pallas as pl


def kernel(indices, weights):
    raise NotImplementedError("write your pallas kernel here")


# speedup vs baseline: 1.5738x; 1.5738x over previous
import functools

import jax
import jax.numpy as jnp
from jax import lax
from jax.experimental import pallas as pl
from jax.experimental.pallas import tpu as pltpu
from jax.experimental.pallas import tpu_sc as plsc

_INFO = plsc.get_sparse_core_info()
_NC, _NS = _INFO.num_cores, _INFO.num_subcores
_NW = _NC * _NS

_B = 16384 * 26
_D = 32
_PER_W = _B // _NW
_CHUNK = 3328
_NCHUNK = _PER_W // _CHUNK

_mesh = plsc.VectorSubcoreMesh(core_axis_name="c", subcore_axis_name="s")


@functools.partial(
    pl.kernel,
    mesh=_mesh,
    compiler_params=pltpu.CompilerParams(use_tc_tiling_on_sc=False),
    out_type=jax.ShapeDtypeStruct((_B, _D), jnp.float32),
    scratch_types=[
        pltpu.VMEM((_CHUNK,), jnp.int32),
        pltpu.VMEM((_CHUNK, _D), jnp.float32),
        pltpu.SemaphoreType.DMA,
    ],
)
def _gather_kernel(idx_hbm, table_hbm, out_hbm, idx_v, rows_v, sem):
    wid = lax.axis_index("s") * _NC + lax.axis_index("c")
    base = wid * _PER_W

    @pl.loop(0, _NCHUNK)
    def _chunk(i):
        off = base + i * _CHUNK
        pltpu.sync_copy(idx_hbm.at[pl.ds(off, _CHUNK)], idx_v)
        pltpu.async_copy(table_hbm.at[idx_v], rows_v, sem).wait()
        pltpu.sync_copy(rows_v, out_hbm.at[pl.ds(off, _CHUNK)])


def kernel(indices, weights):
    flat_idx = indices.reshape(-1).astype(jnp.int32)
    out = _gather_kernel(flat_idx, weights)
    return out.reshape(indices.shape + (weights.shape[-1],))


# speedup vs baseline: 1.5786x; 1.0030x over previous
import functools

import jax
import jax.numpy as jnp
from jax import lax
from jax.experimental import pallas as pl
from jax.experimental.pallas import tpu as pltpu
from jax.experimental.pallas import tpu_sc as plsc

_INFO = plsc.get_sparse_core_info()
_NC, _NS = _INFO.num_cores, _INFO.num_subcores
_NW = _NC * _NS

_B = 16384 * 26
_D = 32
_PER_W = _B // _NW
_CHUNK = 832
_NCHUNK = _PER_W // _CHUNK
_NBUF = 4

_mesh = plsc.VectorSubcoreMesh(core_axis_name="c", subcore_axis_name="s")


@functools.partial(
    pl.kernel,
    mesh=_mesh,
    compiler_params=pltpu.CompilerParams(use_tc_tiling_on_sc=False),
    out_type=jax.ShapeDtypeStruct((_B, _D), jnp.float32),
    scratch_types=[
        pltpu.VMEM((_PER_W,), jnp.int32),
        *[pltpu.VMEM((_CHUNK, _D), jnp.float32) for _ in range(_NBUF)],
        *[pltpu.SemaphoreType.DMA for _ in range(2 * _NBUF)],
    ],
)
def _gather_kernel(idx_hbm, table_hbm, out_hbm, idx_all, *rest):
    bufs = rest[:_NBUF]
    gsem = rest[_NBUF:2 * _NBUF]
    ssem = rest[2 * _NBUF:]
    wid = lax.axis_index("s") * _NC + lax.axis_index("c")
    base = wid * _PER_W

    pltpu.sync_copy(idx_hbm.at[pl.ds(base, _PER_W)], idx_all)

    def start_gather(j, b):
        return pltpu.async_copy(
            table_hbm.at[idx_all.at[pl.ds(j * _CHUNK, _CHUNK)]], bufs[b], gsem[b])

    gathers = [start_gather(j, j) for j in range(_NBUF)]
    gathers += [None] * (_NCHUNK - _NBUF)
    stores = [None] * _NBUF

    for i in range(_NCHUNK):
        b = i % _NBUF
        gathers[i].wait()
        if i > 0:
            j = i - 1 + _NBUF
            if j < _NCHUNK:
                pb = (i - 1) % _NBUF
                stores[pb].wait()
                gathers[j] = start_gather(j, pb)
        stores[b] = pltpu.async_copy(
            bufs[b], out_hbm.at[pl.ds(base + i * _CHUNK, _CHUNK)], ssem[b])

    for i in range(_NCHUNK - _NBUF + 1, _NCHUNK):
        stores[i % _NBUF].wait()
    stores[(_NCHUNK - _NBUF) % _NBUF].wait()


def kernel(indices, weights):
    flat_idx = indices.reshape(-1).astype(jnp.int32)
    out = _gather_kernel(flat_idx, weights)
    return out.reshape(indices.shape + (weights.shape[-1],))
